# baseline (speedup 1.0000x reference)
import functools

import jax
import jax.numpy as jnp
from jax import lax
from jax.experimental import pallas as pl
from jax.experimental.pallas import tpu as pltpu
from jax.experimental.pallas import tpu_sc as plsc

_N = 1024
_CH = 8
_ROW_WORDS = _CH * _N
_RPW = 256


def _sc_body(x_hbm, out_hbm, xbuf, btab, rowbuf, sem):
    cid = lax.axis_index("c")
    sid = lax.axis_index("s")
    wid = sid * 2 + cid
    oc = wid // 4
    j0 = (wid % 4) * _RPW
    row0 = oc * _N + j0

    for ic in range(_CH):
        seg = (oc * _CH + ic) * _N
        pltpu.sync_copy(x_hbm.at[pl.ds(seg, _N)], xbuf)
        base = ic * 2 * _N

        def rev_body(c, _, base=base):
            v = xbuf[pl.ds(1008 - 16 * c, 16)]
            btab[pl.ds(base + 16 * c, 16)] = lax.rev(v, (0,))
            return 0

        lax.fori_loop(0, _N // 16, rev_body, 0)

        def dup_body(c, _, base=base):
            btab[pl.ds(base + _N + 16 * c, 16)] = btab[pl.ds(base + 16 * c, 16)]
            return 0

        lax.fori_loop(0, _N // 16, dup_body, 0)

    def row_body(m, _):
        b = (m & 1) * _ROW_WORDS
        o = 1023 - (j0 + m)

        @pl.when(m >= 2)
        def _wait_prev():
            pltpu.make_async_copy(
                rowbuf.at[pl.ds(b, _ROW_WORDS)],
                out_hbm.at[pl.ds((row0 + m - 2) * _ROW_WORDS, _ROW_WORDS)],
                sem,
            ).wait()

        def chunk(g, _):
            t0 = g * 8
            for u in range(8):
                t = t0 + u
                src = (t >> 6) * (2 * _N) + o + 16 * (t & 63)
                rowbuf[pl.ds(b + 16 * t, 16)] = btab[pl.ds(src, 16)]
            return 0

        lax.fori_loop(0, _ROW_WORDS // (16 * 8), chunk, 0)

        pltpu.async_copy(
            rowbuf.at[pl.ds(b, _ROW_WORDS)],
            out_hbm.at[pl.ds((row0 + m) * _ROW_WORDS, _ROW_WORDS)],
            sem,
        )
        return 0

    lax.fori_loop(0, _RPW, row_body, 0)

    for m in (_RPW - 2, _RPW - 1):
        pltpu.make_async_copy(
            rowbuf.at[pl.ds((m & 1) * _ROW_WORDS, _ROW_WORDS)],
            out_hbm.at[pl.ds((row0 + m) * _ROW_WORDS, _ROW_WORDS)],
            sem,
        ).wait()


@functools.partial(jax.jit, static_argnums=())
def _sc_call(x):
    run = pl.kernel(
        _sc_body,
        out_type=jax.ShapeDtypeStruct((_CH * _N * _CH * _N,), jnp.float32),
        mesh=plsc.VectorSubcoreMesh(core_axis_name="c", subcore_axis_name="s"),
        scratch_types=[
            pltpu.VMEM((_N,), jnp.float32),
            pltpu.VMEM((_CH * 2 * _N,), jnp.float32),
            pltpu.VMEM((2 * _ROW_WORDS,), jnp.float32),
            pltpu.SemaphoreType.DMA,
        ],
    )
    return run(x)


def kernel(X, idx_matrix):
    del idx_matrix
    return _sc_call(X).reshape(_CH * _N, _CH * _N)


# speedup vs baseline: 1.9495x; 1.9495x over previous
import functools

import jax
import jax.numpy as jnp
from jax import lax
from jax.experimental import pallas as pl
from jax.experimental.pallas import tpu as pltpu
from jax.experimental.pallas import tpu_sc as plsc

_N = 1024
_CH = 8
_ROW_WORDS = _CH * _N
_RPW = 256


def _sc_body(x_hbm, out_hbm, xbuf, btab, rowbuf, sem):
    cid = lax.axis_index("c")
    sid = lax.axis_index("s")
    wid = sid * 2 + cid
    oc = wid // 4
    j0 = (wid % 4) * _RPW
    row0 = oc * _N + j0

    for ic in range(_CH):
        seg = (oc * _CH + ic) * _N
        pltpu.sync_copy(x_hbm.at[pl.ds(seg, _N)], xbuf)
        base = ic * 2 * _N

        def rev_body(c, _, base=base):
            v = xbuf[pl.ds(1008 - 16 * c, 16)]
            btab[pl.ds(base + 16 * c, 16)] = lax.rev(v, (0,))
            return 0

        lax.fori_loop(0, _N // 16, rev_body, 0)

        def dup_body(c, _, base=base):
            btab[pl.ds(base + _N + 16 * c, 16)] = btab[pl.ds(base + 16 * c, 16)]
            return 0

        lax.fori_loop(0, _N // 16, dup_body, 0)

    def row_body(m, _):
        b = (m & 1) * _ROW_WORDS
        o = 1023 - (j0 + m)

        @pl.when(m >= 2)
        def _wait_prev():
            pltpu.make_async_copy(
                rowbuf.at[pl.ds(b, _ROW_WORDS)],
                out_hbm.at[pl.ds((row0 + m - 2) * _ROW_WORDS, _ROW_WORDS)],
                sem,
            ).wait()

        for ic in range(_CH):
            src0 = ic * 2 * _N + o
            dst0 = b + ic * _N

            @plsc.parallel_loop(0, _N // 16, 1, unroll=8)
            def chunk(c, src0=src0, dst0=dst0):
                off = 16 * c
                rowbuf[pl.ds(dst0 + off, 16)] = btab[pl.ds(src0 + off, 16)]

        pltpu.async_copy(
            rowbuf.at[pl.ds(b, _ROW_WORDS)],
            out_hbm.at[pl.ds((row0 + m) * _ROW_WORDS, _ROW_WORDS)],
            sem,
        )
        return 0

    lax.fori_loop(0, _RPW, row_body, 0)

    for m in (_RPW - 2, _RPW - 1):
        pltpu.make_async_copy(
            rowbuf.at[pl.ds((m & 1) * _ROW_WORDS, _ROW_WORDS)],
            out_hbm.at[pl.ds((row0 + m) * _ROW_WORDS, _ROW_WORDS)],
            sem,
        ).wait()


@functools.partial(jax.jit, static_argnums=())
def _sc_call(x):
    run = pl.kernel(
        _sc_body,
        out_type=jax.ShapeDtypeStruct((_CH * _N * _CH * _N,), jnp.float32),
        mesh=plsc.VectorSubcoreMesh(core_axis_name="c", subcore_axis_name="s"),
        scratch_types=[
            pltpu.VMEM((_N,), jnp.float32),
            pltpu.VMEM((_CH * 2 * _N,), jnp.float32),
            pltpu.VMEM((2 * _ROW_WORDS,), jnp.float32),
            pltpu.SemaphoreType.DMA,
        ],
    )
    return run(x)


def kernel(X, idx_matrix):
    del idx_matrix
    return _sc_call(X).reshape(_CH * _N, _CH * _N)


# speedup vs baseline: 1.9591x; 1.0050x over previous
import functools

import jax
import jax.numpy as jnp
from jax import lax
from jax.experimental import pallas as pl
from jax.experimental.pallas import tpu as pltpu
from jax.experimental.pallas import tpu_sc as plsc

_N = 1024
_CH = 8
_ROW_WORDS = _CH * _N
_RPW = 256


def _sc_body(x_hbm, out_hbm, xbuf, btab, rowbuf, sem):
    cid = lax.axis_index("c")
    sid = lax.axis_index("s")
    wid = sid * 2 + cid
    oc = wid // 4
    j0 = (wid % 4) * _RPW
    row0 = oc * _N + j0

    for ic in range(_CH):
        seg = (oc * _CH + ic) * _N
        pltpu.sync_copy(x_hbm.at[pl.ds(seg, _N)], xbuf)
        base = ic * 2 * _N

        def rev_body(c, _, base=base):
            v = xbuf[pl.ds(1008 - 16 * c, 16)]
            btab[pl.ds(base + 16 * c, 16)] = lax.rev(v, (0,))
            return 0

        lax.fori_loop(0, _N // 16, rev_body, 0)

        def dup_body(c, _, base=base):
            btab[pl.ds(base + _N + 16 * c, 16)] = btab[pl.ds(base + 16 * c, 16)]
            return 0

        lax.fori_loop(0, _N // 16, dup_body, 0)

    def row_body(m, _):
        b = (m & 1) * _ROW_WORDS
        o = 1023 - (j0 + m)

        @pl.when(m >= 2)
        def _wait_prev():
            pltpu.make_async_copy(
                rowbuf.at[pl.ds(b, _ROW_WORDS)],
                out_hbm.at[pl.ds((row0 + m - 2) * _ROW_WORDS, _ROW_WORDS)],
                sem,
            ).wait()

        for ic in range(_CH):
            src0 = ic * 2 * _N + o
            dst0 = b + ic * _N

            @plsc.parallel_loop(0, _N // 16, 1, unroll=16)
            def chunk(c, src0=src0, dst0=dst0):
                off = 16 * c
                rowbuf[pl.ds(dst0 + off, 16)] = btab[pl.ds(src0 + off, 16)]

        pltpu.async_copy(
            rowbuf.at[pl.ds(b, _ROW_WORDS)],
            out_hbm.at[pl.ds((row0 + m) * _ROW_WORDS, _ROW_WORDS)],
            sem,
        )
        return 0

    lax.fori_loop(0, _RPW, row_body, 0)

    for m in (_RPW - 2, _RPW - 1):
        pltpu.make_async_copy(
            rowbuf.at[pl.ds((m & 1) * _ROW_WORDS, _ROW_WORDS)],
            out_hbm.at[pl.ds((row0 + m) * _ROW_WORDS, _ROW_WORDS)],
            sem,
        ).wait()


@functools.partial(jax.jit, static_argnums=())
def _sc_call(x):
    run = pl.kernel(
        _sc_body,
        out_type=jax.ShapeDtypeStruct((_CH * _N * _CH * _N,), jnp.float32),
        mesh=plsc.VectorSubcoreMesh(core_axis_name="c", subcore_axis_name="s"),
        scratch_types=[
            pltpu.VMEM((_N,), jnp.float32),
            pltpu.VMEM((_CH * 2 * _N,), jnp.float32),
            pltpu.VMEM((2 * _ROW_WORDS,), jnp.float32),
            pltpu.SemaphoreType.DMA,
        ],
    )
    return run(x)


def kernel(X, idx_matrix):
    del idx_matrix
    return _sc_call(X).reshape(_CH * _N, _CH * _N)
